# baseline (speedup 1.0000x reference)
import jax
import jax.numpy as jnp
from jax.experimental import pallas as pl
from jax.experimental.pallas import tpu as pltpu


def _k1_body(x_ref, w_ref, b_ref, o_ref):
    x = x_ref[0].astype(jnp.bfloat16)
    y = jnp.dot(w_ref[...], x, preferred_element_type=jnp.float32)
    o_ref[0] = (y + b_ref[...]).astype(jnp.bfloat16)


def _k1_add_body(x_ref, w_ref, b_ref, s_ref, g_ref, o_ref):
    x = x_ref[0].astype(jnp.bfloat16)
    y = jnp.dot(w_ref[...], x, preferred_element_type=jnp.float32)
    up = jnp.dot(s_ref[0], g_ref[...],
                 preferred_element_type=jnp.float32)
    o_ref[0] = (y + b_ref[...] + up).astype(jnp.bfloat16)


def _upsample_gather(W, TS):
    jj = jnp.arange(TS)
    src = (jj // (2 * W)) * (W // 2) + (jj % W) // 2
    return (src[None, :] == jnp.arange(TS // 4)[:, None]).astype(jnp.bfloat16)


def _lateral(feat, w_oihw, bias, src_flat):
    N, Cin, H, W = feat.shape
    C = w_oihw.shape[0]
    HW = H * W
    x = feat.reshape(N, Cin, HW)
    w2 = w_oihw[:, :, 0, 0].astype(jnp.bfloat16)
    b2 = bias.reshape(C, 1).astype(jnp.float32)

    TS = HW if src_flat is None else min(4096, HW)
    grid = (N, HW // TS)
    in_specs = [
        pl.BlockSpec((1, Cin, TS), lambda n, j: (n, 0, j)),
        pl.BlockSpec((C, Cin), lambda n, j: (0, 0)),
        pl.BlockSpec((C, 1), lambda n, j: (0, 0)),
    ]
    args = [x, w2, b2]
    if src_flat is None:
        body = _k1_body
    else:
        body = _k1_add_body
        in_specs += [
            pl.BlockSpec((1, C, TS // 4), lambda n, j: (n, 0, j)),
            pl.BlockSpec((TS // 4, TS), lambda n, j: (0, 0)),
        ]
        args += [src_flat, _upsample_gather(W, TS)]

    bytes_acc = (N * HW * Cin * 4 + Cin * C * 2 + C * 4 + N * HW * C * 2
                 + (0 if src_flat is None else N * HW // 4 * C * 2))
    out = pl.pallas_call(
        body,
        out_shape=jax.ShapeDtypeStruct((N, C, HW), jnp.bfloat16),
        grid=grid,
        in_specs=in_specs,
        out_specs=pl.BlockSpec((1, C, TS), lambda n, j: (n, 0, j)),
        compiler_params=pltpu.CompilerParams(
            dimension_semantics=("parallel", "parallel"),
            vmem_limit_bytes=64 * 1024 * 1024,
        ),
        cost_estimate=pl.CostEstimate(
            flops=int(2 * N * HW * Cin * C), transcendentals=0,
            bytes_accessed=int(bytes_acc)),
    )(*args)
    return out


def _make_k2_body(TH, W, Ht, row_halo):
    S = TH * W

    def _body(xp_ref, xc_ref, xn_ref, w_ref, b_ref, o_ref):
        i = pl.program_id(1)
        C = xc_ref.shape[1]
        dt = xc_ref.dtype
        prev = xp_ref[0] if row_halo else xp_ref[0, :, S - W:]
        nxt = xn_ref[0] if row_halo else xn_ref[0, :, :W]
        top = jnp.where(i > 0, prev, jnp.zeros((C, W), dt))
        bot = jnp.where(i < Ht - 1, nxt, jnp.zeros((C, W), dt))
        xfull = jnp.concatenate([top, xc_ref[0], bot], axis=1)
        lane = jax.lax.broadcasted_iota(jnp.int32, (1, S + 2 * W), 1) % W
        zc = jnp.zeros((C, 1), dt)
        sL = jnp.concatenate([zc, xfull[:, :-1]], axis=1)
        sL = jnp.where(lane == 0, jnp.zeros((), dt), sL)
        sR = jnp.concatenate([xfull[:, 1:], zc], axis=1)
        sR = jnp.where(lane == W - 1, jnp.zeros((), dt), sR)
        srcs = (sL, xfull, sR)
        acc = None
        for dy in range(3):
            for dx in range(3):
                op = srcs[dx][:, dy * W: dy * W + S]
                d = jnp.dot(w_ref[3 * dy + dx], op,
                            preferred_element_type=jnp.float32)
                acc = d if acc is None else acc + d
        o_ref[0] = acc + b_ref[...]
    return _body


def _k2_row_tile(H, W):
    best = 1
    for th in range(1, H + 1):
        if H % th == 0 and th * W <= 4096 and H // th >= 2:
            best = th
    return best


def _smooth(inner_flat, w_oihw, bias, N, H, W):
    C = w_oihw.shape[0]
    TH = _k2_row_tile(H, W)
    Ht = H // TH
    w9 = jnp.transpose(w_oihw, (2, 3, 0, 1)).reshape(9, C, C)
    w9 = w9.astype(jnp.bfloat16)
    b2 = bias.reshape(C, 1).astype(jnp.float32)

    row_halo = (W % 128 == 0)
    if row_halo:
        in_specs = [
            pl.BlockSpec((1, C, W),
                         lambda n, i: (n, 0, jnp.clip(i * TH - 1, 0, H - 1))),
            pl.BlockSpec((1, C, TH * W), lambda n, i: (n, 0, i)),
            pl.BlockSpec((1, C, W),
                         lambda n, i: (n, 0, jnp.clip((i + 1) * TH, 0, H - 1))),
        ]
    else:
        in_specs = [
            pl.BlockSpec((1, C, TH * W),
                         lambda n, i, _k=k: (n, 0, jnp.clip(i + _k, 0, Ht - 1)))
            for k in (-1, 0, 1)
        ]
    in_specs += [
        pl.BlockSpec((9, C, C), lambda n, i: (0, 0, 0)),
        pl.BlockSpec((C, 1), lambda n, i: (0, 0)),
    ]
    flops = 2 * N * H * W * 9 * C * C
    bytes_acc = (N * (H + 2 * Ht) * W * C * 2 + 9 * C * C * 2 + C * 4
                 + N * H * W * C * 4)
    out = pl.pallas_call(
        _make_k2_body(TH, W, Ht, row_halo),
        out_shape=jax.ShapeDtypeStruct((N, C, H * W), jnp.float32),
        grid=(N, Ht),
        in_specs=in_specs,
        out_specs=pl.BlockSpec((1, C, TH * W), lambda n, i: (n, 0, i)),
        compiler_params=pltpu.CompilerParams(
            dimension_semantics=("parallel", "parallel"),
            vmem_limit_bytes=64 * 1024 * 1024,
        ),
        cost_estimate=pl.CostEstimate(
            flops=int(flops), transcendentals=0,
            bytes_accessed=int(bytes_acc)),
    )(inner_flat, inner_flat, inner_flat, w9, b2)
    return out.reshape(N, C, H, W)


def kernel(feat0, feat1, feat2, feat3,
           inner_w0, inner_b0, layer_w0, layer_b0,
           inner_w1, inner_b1, layer_w1, layer_b1,
           inner_w2, inner_b2, layer_w2, layer_b2,
           inner_w3, inner_b3, layer_w3, layer_b3):
    feats = [feat0, feat1, feat2, feat3]
    iw = [inner_w0, inner_w1, inner_w2, inner_w3]
    ib = [inner_b0, inner_b1, inner_b2, inner_b3]
    lw = [layer_w0, layer_w1, layer_w2, layer_w3]
    lb = [layer_b0, layer_b1, layer_b2, layer_b3]

    names = ["feat0", "feat1", "feat2", "feat3"]
    results = [None] * 4
    last_inner = None
    for idx in range(3, -1, -1):
        N, _, H, W = feats[idx].shape
        last_inner = _lateral(feats[idx], iw[idx], ib[idx], last_inner)
        results[idx] = _smooth(last_inner, lw[idx], lb[idx], N, H, W)

    from collections import OrderedDict
    return OrderedDict(zip(names, results))
